# baseline (speedup 1.0000x reference)
import jax
import jax.numpy as jnp
from jax import lax
from jax.experimental import pallas as pl
from jax.experimental.pallas import tpu as pltpu
from jax.experimental.pallas import tpu_sc as plsc

N = 10000
E = 320000
D = 128
NCORE = 2
NSUB = 16
NWORK = NCORE * NSUB
CH = 128
N_PAD = 10240
H = N_PAD // 2
XROWS_PT = H // NSUB
ACC_ROWS = H + CH
ACC_PT = ACC_ROWS // NSUB
GRAN = NSUB * CH
CAPCH = ((E + GRAN - 1) // GRAN * GRAN // CH + NSUB - 1) // NSUB * NSUB
CAP = CAPCH * CH

NCHUNK = 80
EPW = CH * NCHUNK
E_PAD = EPW * NWORK

_MESH = plsc.VectorSubcoreMesh(core_axis_name="c", subcore_axis_name="s",
                               num_cores=NCORE, num_subcores=NSUB)


def _agg_body(x_hbm, idx_hbm, cnt_hbm, out_hbm,
              zb, cnt_v, comb0, comb1, rows, x_sh, acc_sh, is0, is1, gsem, zsem):
    c = lax.axis_index("c")
    s = lax.axis_index("s")
    z16 = jnp.zeros((16,), jnp.float32)
    for i in range(16):
        for j in range(8):
            zb[i, pl.ds(j * 16, 16)] = z16
    pltpu.sync_copy(cnt_hbm.at[c], cnt_v)
    cnts = cnt_v[:]
    pltpu.async_copy(x_hbm.at[pl.ds(c * H + s * XROWS_PT, XROWS_PT)],
                     x_sh.at[pl.ds(s * XROWS_PT, XROWS_PT)], gsem)
    pltpu.make_async_copy(x_hbm.at[pl.ds(c * H + s * XROWS_PT, XROWS_PT)],
                          x_sh.at[pl.ds(s * XROWS_PT, XROWS_PT)], gsem).wait()

    for p in (0, 1):
        b = c * 2 + p
        n16 = cnts[p]
        rbase = s * ACC_PT
        for k in range(ACC_PT // 16):
            pltpu.async_copy(zb, acc_sh.at[pl.ds(rbase + k * 16, 16)], zsem)
        pltpu.async_copy(zb.at[pl.ds(0, ACC_PT % 16)],
                         acc_sh.at[pl.ds(rbase + ACC_PT - ACC_PT % 16,
                                         ACC_PT % 16)], zsem)
        for k in range(ACC_PT // 16):
            pltpu.make_async_copy(zb, acc_sh.at[pl.ds(rbase + k * 16, 16)],
                                  zsem).wait()
        pltpu.make_async_copy(zb.at[pl.ds(0, ACC_PT % 16)],
                              acc_sh.at[pl.ds(rbase + ACC_PT - ACC_PT % 16,
                                              ACC_PT % 16)], zsem).wait()
        plsc.subcore_barrier()

        row0 = 2 * CAPCH * b + 2 * s * n16

        def edge_body(j, carry):
            pltpu.sync_copy(idx_hbm.at[pl.ds(row0 + 2 * j, 2)], comb0)
            pltpu.async_copy(x_sh.at[comb0.at[0]], rows, gsem).wait()
            pltpu.sync_copy(rows, acc_sh.at[comb0.at[1]], add=True)
            return carry

        lax.fori_loop(0, n16, edge_body, 0)
        plsc.subcore_barrier()
        pltpu.sync_copy(acc_sh.at[pl.ds(s * XROWS_PT, XROWS_PT)],
                        out_hbm.at[c, pl.ds(p * H + s * XROWS_PT, XROWS_PT)])


_agg_call = pl.kernel(
    _agg_body,
    out_type=jax.ShapeDtypeStruct((NCORE, N_PAD, D), jnp.float32),
    mesh=_MESH,
    scratch_types=[
        pltpu.VMEM((16, D), jnp.float32),
        pltpu.VMEM((16,), jnp.int32),
        pltpu.VMEM((2, CH), jnp.int32),
        pltpu.VMEM((2, CH), jnp.int32),
        pltpu.VMEM((CH, D), jnp.float32),
        pltpu.VMEM_SHARED((H, D), jnp.float32),
        pltpu.VMEM_SHARED((ACC_ROWS, D), jnp.float32),
        pltpu.SemaphoreType.DMA,
        pltpu.SemaphoreType.DMA,
        pltpu.SemaphoreType.DMA,
        pltpu.SemaphoreType.DMA,
    ],
)


def _deg_body(dst_hbm, out_hbm, ones_v, zb, dst_v, dacc_sh):
    c = lax.axis_index("c")
    s = lax.axis_index("s")
    wid = c * NSUB + s
    one_row = jnp.where(lax.iota(jnp.int32, 16) == 0,
                        jnp.float32(1.0), jnp.float32(0.0))
    z16 = jnp.zeros((16,), jnp.float32)
    for i in range(CH):
        for j in range(8):
            ones_v[i, pl.ds(j * 16, 16)] = one_row if j == 0 else z16
    for i in range(16):
        for j in range(8):
            zb[i, pl.ds(j * 16, 16)] = z16
    rbase = s * (N_PAD // NSUB)

    def zero_body(k, carry):
        pltpu.sync_copy(zb, dacc_sh.at[pl.ds(rbase + k * 16, 16)])
        return carry

    lax.fori_loop(0, N_PAD // NSUB // 16, zero_body, 0)
    ebase = wid * EPW
    plsc.subcore_barrier()

    def edge_body(j, carry):
        pltpu.sync_copy(dst_hbm.at[pl.ds(ebase + j * CH, CH)], dst_v)
        pltpu.sync_copy(ones_v, dacc_sh.at[dst_v], add=True)
        return carry

    lax.fori_loop(0, NCHUNK, edge_body, 0)
    plsc.subcore_barrier()
    pltpu.sync_copy(dacc_sh.at[pl.ds(rbase, N_PAD // NSUB)],
                    out_hbm.at[c, pl.ds(rbase, N_PAD // NSUB)])


_deg_call = pl.kernel(
    _deg_body,
    out_type=jax.ShapeDtypeStruct((NCORE, N_PAD, D), jnp.float32),
    mesh=_MESH,
    scratch_types=[
        pltpu.VMEM((CH, D), jnp.float32),
        pltpu.VMEM((16, D), jnp.float32),
        pltpu.VMEM((CH,), jnp.int32),
        pltpu.VMEM_SHARED((N_PAD, D), jnp.float32),
    ],
)


def _embed_body(h_ref, emb_ref, out_ref):
    hh = h_ref[:]
    cols = lax.broadcasted_iota(jnp.int32, (1, D), 1)
    onehot = (hh == cols).astype(jnp.float32)
    out_ref[:] = jnp.dot(onehot, emb_ref[:], preferred_element_type=jnp.float32,
                         precision=lax.Precision.HIGHEST)


_embed_call = pl.pallas_call(
    _embed_body,
    out_shape=jax.ShapeDtypeStruct((N_PAD, D), jnp.float32),
)


def _dense_common(sacc_ref, dacc_ref, x_ref, wlt_ref, bl_ref, wrt_ref, g_ref, be_ref):
    ssum = sacc_ref[0] + sacc_ref[1]
    deg = (dacc_ref[0] + dacc_ref[1])[:, 0:1]
    rdeg = 1.0 / jnp.maximum(deg, 1.0)
    mean = ssum * rdeg
    x = x_ref[:]
    t = (jnp.dot(mean, wlt_ref[:], preferred_element_type=jnp.float32)
         + bl_ref[:]
         + jnp.dot(x, wrt_ref[:], preferred_element_type=jnp.float32))
    tv = t[:N]
    mu = jnp.mean(tv, axis=0, keepdims=True)
    var = jnp.mean((tv - mu) * (tv - mu), axis=0, keepdims=True)
    return g_ref[:] * (t - mu) * lax.rsqrt(var + 1e-5) + be_ref[:] + x


def _dense_body(sacc_ref, dacc_ref, x_ref, wlt_ref, bl_ref, wrt_ref, g_ref, be_ref,
                out_ref):
    out_ref[:] = _dense_common(sacc_ref, dacc_ref, x_ref, wlt_ref, bl_ref,
                               wrt_ref, g_ref, be_ref)


_dense_call = pl.pallas_call(
    _dense_body,
    out_shape=jax.ShapeDtypeStruct((N_PAD, D), jnp.float32),
)


def _final_body(sacc_ref, dacc_ref, x_ref, wlt_ref, bl_ref, wrt_ref, g_ref, be_ref,
                wm0_ref, bm0_ref, wm1_ref, bm1_ref, wm2_ref, bm2_ref, out_ref):
    xo = _dense_common(sacc_ref, dacc_ref, x_ref, wlt_ref, bl_ref, wrt_ref,
                       g_ref, be_ref)[:N]
    y = jnp.maximum(jnp.dot(xo, wm0_ref[:], preferred_element_type=jnp.float32)
                    + bm0_ref[:], 0.0)
    y = jnp.maximum(jnp.dot(y, wm1_ref[:], preferred_element_type=jnp.float32)
                    + bm1_ref[:], 0.0)
    out_ref[:] = jnp.dot(y, wm2_ref[:], preferred_element_type=jnp.float32) + bm2_ref[:]


_final_call = pl.pallas_call(
    _final_body,
    out_shape=jax.ShapeDtypeStruct((N, D), jnp.float32),
)


def _pad_mat(w_t, rows, cols):
    return jnp.zeros((rows, cols), jnp.float32).at[: w_t.shape[0], : w_t.shape[1]].set(w_t)


def _pad_vec(b, cols):
    return jnp.zeros((1, cols), jnp.float32).at[0, : b.shape[0]].set(b)


def kernel(h, edge_index, e, emb,
           Wl0, bl0, Wr0, g0, be0,
           Wl1, bl1, Wr1, g1, be1,
           Wl2, bl2, Wr2, g2, be2,
           Wl3, bl3, Wr3, g3, be3,
           Wm0, bm0, Wm1, bm1, Wm2, bm2):
    src = edge_index[0].astype(jnp.int32)
    dst = edge_index[1].astype(jnp.int32)

    pad = E_PAD - E
    dst_p = jnp.concatenate([dst, jnp.full((pad,), N, jnp.int32)])

    bs = (src >= H).astype(jnp.int32) * 2 + (dst >= H).astype(jnp.int32)
    b_s, src_s, dst_s = lax.sort((bs, src, dst), num_keys=1, is_stable=True)
    counts = jnp.sum(b_s[None, :] == jnp.arange(4, dtype=jnp.int32)[:, None],
                     axis=1)
    cumoff = jnp.concatenate([jnp.zeros((1,), jnp.int32),
                              jnp.cumsum(counts)[:3]])
    nk16 = (counts + GRAN - 1) // GRAN
    cnt16 = jnp.zeros((2, 16), jnp.int32).at[:, :2].set(nk16.reshape(2, 2))
    q = jnp.arange(4 * CAP, dtype=jnp.int32)
    k = q // CAP
    j = q % CAP
    take = j < counts[k]
    gidx = jnp.clip(cumoff[k] + j, 0, E - 1)
    src_l = jnp.where(take, src_s[gidx] - (k // 2) * H, 0)
    dst_l = jnp.where(take, dst_s[gidx] - (k % 2) * H, H)
    idx2d = jnp.stack([src_l.reshape(4 * CAPCH, CH),
                       dst_l.reshape(4 * CAPCH, CH)], axis=1).reshape(
                           8 * CAPCH, CH)

    h_p = jnp.concatenate([h.astype(jnp.int32),
                           jnp.zeros((N_PAD - N,), jnp.int32)])[:, None]
    x = _embed_call(h_p, emb)
    dacc = _deg_call(dst_p)

    layers = [(Wl0, bl0, Wr0, g0, be0), (Wl1, bl1, Wr1, g1, be1),
              (Wl2, bl2, Wr2, g2, be2)]
    for (Wl, bl, Wr, g, be) in layers:
        sacc = _agg_call(x, idx2d, cnt16)
        x = _dense_call(sacc, dacc, x, Wl.T, bl[None, :], Wr.T, g[None, :],
                        be[None, :])

    sacc = _agg_call(x, idx2d, cnt16)
    y = _final_call(sacc, dacc, x, Wl3.T, bl3[None, :], Wr3.T, g3[None, :],
                    be3[None, :],
                    _pad_mat(Wm0.T, D, D), _pad_vec(bm0, D),
                    _pad_mat(Wm1.T, D, D), _pad_vec(bm1, D),
                    _pad_mat(Wm2.T, D, D), _pad_vec(bm2, D))
    return y[:, : bm2.shape[0]]


# speedup vs baseline: 23.2049x; 23.2049x over previous
import jax
import jax.numpy as jnp
from jax import lax
from jax.experimental import pallas as pl
from jax.experimental.pallas import tpu as pltpu
from jax.experimental.pallas import tpu_sc as plsc

N = 10000
E = 320000
D = 128
NCORE = 2
NSUB = 16
NWORK = NCORE * NSUB
CH = 128
N_PAD = 10240
H = N_PAD // 2
XROWS_PT = H // NSUB
ACC_ROWS = H + CH
ACC_PT = ACC_ROWS // NSUB
GRAN = NSUB * CH
CAPCH = ((E + GRAN - 1) // GRAN * GRAN // CH + NSUB - 1) // NSUB * NSUB
CAP = CAPCH * CH

NCHUNK = 80
EPW = CH * NCHUNK
E_PAD = EPW * NWORK

_MESH = plsc.VectorSubcoreMesh(core_axis_name="c", subcore_axis_name="s",
                               num_cores=NCORE, num_subcores=NSUB)


def _agg_body(x_hbm, idx_hbm, cnt_hbm, out_hbm,
              zb, cnt_v, comb0, comb1, rows, x_sh, acc_sh, is0, is1, gsem, zsem):
    c = lax.axis_index("c")
    s = lax.axis_index("s")
    z16 = jnp.zeros((16,), jnp.float32)
    for i in range(16):
        for j in range(8):
            zb[i, pl.ds(j * 16, 16)] = z16
    pltpu.sync_copy(cnt_hbm.at[c], cnt_v)
    cnts = cnt_v[:]
    pltpu.async_copy(x_hbm.at[pl.ds(c * H + s * XROWS_PT, XROWS_PT)],
                     x_sh.at[pl.ds(s * XROWS_PT, XROWS_PT)], gsem)
    pltpu.make_async_copy(x_hbm.at[pl.ds(c * H + s * XROWS_PT, XROWS_PT)],
                          x_sh.at[pl.ds(s * XROWS_PT, XROWS_PT)], gsem).wait()

    for p in (0, 1):
        b = c * 2 + p
        n16 = cnts[p]
        rbase = s * ACC_PT
        for k in range(ACC_PT // 16):
            pltpu.async_copy(zb, acc_sh.at[pl.ds(rbase + k * 16, 16)], zsem)
        pltpu.async_copy(zb.at[pl.ds(0, ACC_PT % 16)],
                         acc_sh.at[pl.ds(rbase + ACC_PT - ACC_PT % 16,
                                         ACC_PT % 16)], zsem)
        for k in range(ACC_PT // 16):
            pltpu.make_async_copy(zb, acc_sh.at[pl.ds(rbase + k * 16, 16)],
                                  zsem).wait()
        pltpu.make_async_copy(zb.at[pl.ds(0, ACC_PT % 16)],
                              acc_sh.at[pl.ds(rbase + ACC_PT - ACC_PT % 16,
                                              ACC_PT % 16)], zsem).wait()
        plsc.subcore_barrier()

        row0 = 2 * CAPCH * b + 2 * s * n16

        def edge_body(j, carry):
            pltpu.sync_copy(idx_hbm.at[pl.ds(row0 + 2 * j, 2)], comb0)
            pltpu.async_copy(x_sh.at[comb0.at[0]], rows, gsem).wait()
            pltpu.sync_copy(rows, acc_sh.at[comb0.at[1]], add=True)
            return carry

        lax.fori_loop(0, n16, edge_body, 0)
        plsc.subcore_barrier()
        pltpu.sync_copy(acc_sh.at[pl.ds(s * XROWS_PT, XROWS_PT)],
                        out_hbm.at[c, pl.ds(p * H + s * XROWS_PT, XROWS_PT)])


_agg_call = pl.kernel(
    _agg_body,
    out_type=jax.ShapeDtypeStruct((NCORE, N_PAD, D), jnp.float32),
    mesh=_MESH,
    scratch_types=[
        pltpu.VMEM((16, D), jnp.float32),
        pltpu.VMEM((16,), jnp.int32),
        pltpu.VMEM((2, CH), jnp.int32),
        pltpu.VMEM((2, CH), jnp.int32),
        pltpu.VMEM((CH, D), jnp.float32),
        pltpu.VMEM_SHARED((H, D), jnp.float32),
        pltpu.VMEM_SHARED((ACC_ROWS, D), jnp.float32),
        pltpu.SemaphoreType.DMA,
        pltpu.SemaphoreType.DMA,
        pltpu.SemaphoreType.DMA,
        pltpu.SemaphoreType.DMA,
    ],
)


def _deg_body(dst_hbm, out_hbm, ones_v, zb, dst_v, dacc_sh):
    c = lax.axis_index("c")
    s = lax.axis_index("s")
    wid = c * NSUB + s
    one_row = jnp.where(lax.iota(jnp.int32, 16) == 0,
                        jnp.float32(1.0), jnp.float32(0.0))
    z16 = jnp.zeros((16,), jnp.float32)
    for i in range(CH):
        for j in range(8):
            ones_v[i, pl.ds(j * 16, 16)] = one_row if j == 0 else z16
    for i in range(16):
        for j in range(8):
            zb[i, pl.ds(j * 16, 16)] = z16
    rbase = s * (N_PAD // NSUB)

    def zero_body(k, carry):
        pltpu.sync_copy(zb, dacc_sh.at[pl.ds(rbase + k * 16, 16)])
        return carry

    lax.fori_loop(0, N_PAD // NSUB // 16, zero_body, 0)
    ebase = wid * EPW
    plsc.subcore_barrier()

    def edge_body(j, carry):
        pltpu.sync_copy(dst_hbm.at[pl.ds(ebase + j * CH, CH)], dst_v)
        pltpu.sync_copy(ones_v, dacc_sh.at[dst_v], add=True)
        return carry

    lax.fori_loop(0, NCHUNK, edge_body, 0)
    plsc.subcore_barrier()
    pltpu.sync_copy(dacc_sh.at[pl.ds(rbase, N_PAD // NSUB)],
                    out_hbm.at[c, pl.ds(rbase, N_PAD // NSUB)])


_deg_call = pl.kernel(
    _deg_body,
    out_type=jax.ShapeDtypeStruct((NCORE, N_PAD, D), jnp.float32),
    mesh=_MESH,
    scratch_types=[
        pltpu.VMEM((CH, D), jnp.float32),
        pltpu.VMEM((16, D), jnp.float32),
        pltpu.VMEM((CH,), jnp.int32),
        pltpu.VMEM_SHARED((N_PAD, D), jnp.float32),
    ],
)


def _embed_body(h_ref, emb_ref, out_ref):
    hh = h_ref[:]
    cols = lax.broadcasted_iota(jnp.int32, (1, D), 1)
    onehot = (hh == cols).astype(jnp.float32)
    out_ref[:] = jnp.dot(onehot, emb_ref[:], preferred_element_type=jnp.float32,
                         precision=lax.Precision.HIGHEST)


_embed_call = pl.pallas_call(
    _embed_body,
    out_shape=jax.ShapeDtypeStruct((N_PAD, D), jnp.float32),
)


def _dense_common(sacc_ref, dacc_ref, x_ref, wlt_ref, bl_ref, wrt_ref, g_ref, be_ref):
    ssum = sacc_ref[0] + sacc_ref[1]
    deg = (dacc_ref[0] + dacc_ref[1])[:, 0:1]
    rdeg = 1.0 / jnp.maximum(deg, 1.0)
    mean = ssum * rdeg
    x = x_ref[:]
    t = (jnp.dot(mean, wlt_ref[:], preferred_element_type=jnp.float32)
         + bl_ref[:]
         + jnp.dot(x, wrt_ref[:], preferred_element_type=jnp.float32))
    tv = t[:N]
    mu = jnp.mean(tv, axis=0, keepdims=True)
    var = jnp.mean((tv - mu) * (tv - mu), axis=0, keepdims=True)
    return g_ref[:] * (t - mu) * lax.rsqrt(var + 1e-5) + be_ref[:] + x


def _dense_body(sacc_ref, dacc_ref, x_ref, wlt_ref, bl_ref, wrt_ref, g_ref, be_ref,
                out_ref):
    out_ref[:] = _dense_common(sacc_ref, dacc_ref, x_ref, wlt_ref, bl_ref,
                               wrt_ref, g_ref, be_ref)


_dense_call = pl.pallas_call(
    _dense_body,
    out_shape=jax.ShapeDtypeStruct((N_PAD, D), jnp.float32),
)


def _final_body(sacc_ref, dacc_ref, x_ref, wlt_ref, bl_ref, wrt_ref, g_ref, be_ref,
                wm0_ref, bm0_ref, wm1_ref, bm1_ref, wm2_ref, bm2_ref, out_ref):
    xo = _dense_common(sacc_ref, dacc_ref, x_ref, wlt_ref, bl_ref, wrt_ref,
                       g_ref, be_ref)[:N]
    y = jnp.maximum(jnp.dot(xo, wm0_ref[:], preferred_element_type=jnp.float32)
                    + bm0_ref[:], 0.0)
    y = jnp.maximum(jnp.dot(y, wm1_ref[:], preferred_element_type=jnp.float32)
                    + bm1_ref[:], 0.0)
    out_ref[:] = jnp.dot(y, wm2_ref[:], preferred_element_type=jnp.float32) + bm2_ref[:]


_final_call = pl.pallas_call(
    _final_body,
    out_shape=jax.ShapeDtypeStruct((N, D), jnp.float32),
)


def _pad_mat(w_t, rows, cols):
    return jnp.zeros((rows, cols), jnp.float32).at[: w_t.shape[0], : w_t.shape[1]].set(w_t)


def _pad_vec(b, cols):
    return jnp.zeros((1, cols), jnp.float32).at[0, : b.shape[0]].set(b)


def kernel(h, edge_index, e, emb,
           Wl0, bl0, Wr0, g0, be0,
           Wl1, bl1, Wr1, g1, be1,
           Wl2, bl2, Wr2, g2, be2,
           Wl3, bl3, Wr3, g3, be3,
           Wm0, bm0, Wm1, bm1, Wm2, bm2):
    src = edge_index[0].astype(jnp.int32)
    dst = edge_index[1].astype(jnp.int32)

    pad = E_PAD - E
    dst_p = jnp.concatenate([dst, jnp.full((pad,), N, jnp.int32)])

    bs = (src >= H).astype(jnp.int32) * 2 + (dst >= H).astype(jnp.int32)
    onehot = (bs[:, None] == jnp.arange(4, dtype=jnp.int32)[None, :]).astype(
        jnp.int32)
    ranks = jnp.cumsum(onehot, axis=0) - 1
    rank = jnp.take_along_axis(ranks, bs[:, None], axis=1)[:, 0]
    counts = ranks[-1] + 1
    nk16 = (counts + GRAN - 1) // GRAN
    cnt16 = jnp.zeros((2, 16), jnp.int32).at[:, :2].set(nk16.reshape(2, 2))
    pos = bs * CAP + rank
    src_l = jnp.zeros((4 * CAP,), jnp.int32).at[pos].set(
        src - (bs // 2) * H, unique_indices=True)
    dst_l = jnp.full((4 * CAP,), H, jnp.int32).at[pos].set(
        dst - (bs % 2) * H, unique_indices=True)
    idx2d = jnp.stack([src_l.reshape(4 * CAPCH, CH),
                       dst_l.reshape(4 * CAPCH, CH)], axis=1).reshape(
                           8 * CAPCH, CH)

    h_p = jnp.concatenate([h.astype(jnp.int32),
                           jnp.zeros((N_PAD - N,), jnp.int32)])[:, None]
    x = _embed_call(h_p, emb)
    dacc = _deg_call(dst_p)

    layers = [(Wl0, bl0, Wr0, g0, be0), (Wl1, bl1, Wr1, g1, be1),
              (Wl2, bl2, Wr2, g2, be2)]
    for (Wl, bl, Wr, g, be) in layers:
        sacc = _agg_call(x, idx2d, cnt16)
        x = _dense_call(sacc, dacc, x, Wl.T, bl[None, :], Wr.T, g[None, :],
                        be[None, :])

    sacc = _agg_call(x, idx2d, cnt16)
    y = _final_call(sacc, dacc, x, Wl3.T, bl3[None, :], Wr3.T, g3[None, :],
                    be3[None, :],
                    _pad_mat(Wm0.T, D, D), _pad_vec(bm0, D),
                    _pad_mat(Wm1.T, D, D), _pad_vec(bm1, D),
                    _pad_mat(Wm2.T, D, D), _pad_vec(bm2, D))
    return y[:, : bm2.shape[0]]
